# baseline (speedup 1.0000x reference)
import functools

import jax
import jax.numpy as jnp
from jax import lax
from jax.experimental import pallas as pl
from jax.experimental.pallas import tpu as pltpu
from jax.experimental.pallas import tpu_sc as plsc

_E = 8
_NEG_INF = -1e30
_D = 768
_DAUG = 896
_N = 8192
_BLK = 512
_NB_MAX = (2 * _N) // _BLK + _E
_R_MAX = _NB_MAX * _BLK
_NW = 32
_TPW = _N // _NW
_CH = 64


def _vgather(vec, idx):
    return lax.gather(
        vec, idx[:, None],
        lax.GatherDimensionNumbers(offset_dims=(),
                                   collapsed_slice_dims=(0,),
                                   start_index_map=(0,)),
        slice_sizes=(1,),
        mode=lax.GatherScatterMode.PROMISE_IN_BOUNDS)


def _gate_body(x_ref, Wg_ref, bg_ref, xw_ref, e1_ref, e2_ref, r1_ref,
               r2_ref, cnt_ref, aux_ref, carry, probs_acc, cnt_acc,
               *, blk, n_tokens):
    t = pl.program_id(0)
    nt = pl.num_programs(0)

    xb = x_ref[...]
    logits = lax.dot_general(
        xb, Wg_ref[...], (((1,), (0,)), ((), ())),
        preferred_element_type=jnp.float32) + bg_ref[...]
    iota_e = lax.broadcasted_iota(jnp.int32, (blk, _E), 1)
    max1 = jnp.max(logits, axis=1, keepdims=True)
    idx1 = jnp.min(jnp.where(logits == max1, iota_e, _E), axis=1,
                   keepdims=True)
    oh1 = (iota_e == idx1).astype(jnp.float32)
    masked = jnp.where(iota_e == idx1, _NEG_INF, logits)
    max2 = jnp.max(masked, axis=1, keepdims=True)
    idx2 = jnp.min(jnp.where(masked == max2, iota_e, _E), axis=1,
                   keepdims=True)
    oh2 = (iota_e == idx2).astype(jnp.float32)
    e2v = jnp.exp(max2 - max1)
    w1 = 1.0 / (1.0 + e2v)
    w2 = 1.0 - w1

    probs = jnp.exp(logits - max1)
    probs = probs / jnp.sum(probs, axis=1, keepdims=True)
    block_probs = jnp.sum(probs, axis=0, keepdims=True)
    block_cnt = jnp.sum(oh1, axis=0, keepdims=True)

    m = oh1 + oh2
    row_i = lax.broadcasted_iota(jnp.int32, (blk, blk), 0)
    col_i = lax.broadcasted_iota(jnp.int32, (blk, blk), 1)
    tril = (col_i < row_i).astype(jnp.float32)
    excl = lax.dot_general(tril, m, (((1,), (0,)), ((), ())),
                           preferred_element_type=jnp.float32)

    @pl.when(t == 0)
    def _init():
        probs_acc[...] = block_probs
        cnt_acc[...] = block_cnt
        carry[...] = jnp.zeros_like(carry)

    @pl.when(t > 0)
    def _accum():
        probs_acc[...] += block_probs
        cnt_acc[...] += block_cnt

    base = excl + carry[...]
    r1 = jnp.sum(base * oh1, axis=1, keepdims=True)
    r2 = jnp.sum(base * oh2, axis=1, keepdims=True)
    carry[...] += jnp.sum(m, axis=0, keepdims=True)

    e1_ref[...] = idx1
    e2_ref[...] = idx2
    r1_ref[...] = r1.astype(jnp.int32)
    r2_ref[...] = r2.astype(jnp.int32)

    pad = jnp.zeros((blk, _DAUG - _D - 1), jnp.float32)
    xw_ref[0] = jnp.concatenate([xb * w1, w1, pad], axis=1)
    xw_ref[1] = jnp.concatenate([xb * w2, w2, pad], axis=1)

    @pl.when(t == nt - 1)
    def _fin():
        cnt_ref[...] = jnp.concatenate(
            [carry[...].astype(jnp.int32),
             jnp.zeros((1, 8), jnp.int32)], axis=1)
        aux_ref[...] = jnp.sum(
            cnt_acc[...] / (n_tokens + 1e-8) * (probs_acc[...] / n_tokens),
            axis=1, keepdims=True) * _E


def _gate(x, Wg, bg):
    n, d = x.shape
    blk = 1024
    body = functools.partial(_gate_body, blk=blk, n_tokens=n)
    return pl.pallas_call(
        body,
        grid=(n // blk,),
        in_specs=[
            pl.BlockSpec((blk, d), lambda t: (t, 0)),
            pl.BlockSpec((d, _E), lambda t: (0, 0)),
            pl.BlockSpec((1, _E), lambda t: (0, 0)),
        ],
        out_specs=[
            pl.BlockSpec((2, blk, _DAUG), lambda t: (0, t, 0)),
            pl.BlockSpec((blk, 1), lambda t: (t, 0)),
            pl.BlockSpec((blk, 1), lambda t: (t, 0)),
            pl.BlockSpec((blk, 1), lambda t: (t, 0)),
            pl.BlockSpec((blk, 1), lambda t: (t, 0)),
            pl.BlockSpec((1, 16), lambda t: (0, 0)),
            pl.BlockSpec((1, 1), lambda t: (0, 0)),
        ],
        out_shape=[
            jax.ShapeDtypeStruct((2, n, _DAUG), jnp.float32),
            jax.ShapeDtypeStruct((n, 1), jnp.int32),
            jax.ShapeDtypeStruct((n, 1), jnp.int32),
            jax.ShapeDtypeStruct((n, 1), jnp.int32),
            jax.ShapeDtypeStruct((n, 1), jnp.int32),
            jax.ShapeDtypeStruct((1, 16), jnp.int32),
            jax.ShapeDtypeStruct((1, 1), jnp.float32),
        ],
        scratch_shapes=[
            pltpu.VMEM((1, _E), jnp.float32),
            pltpu.VMEM((1, _E), jnp.float32),
            pltpu.VMEM((1, _E), jnp.float32),
        ],
        compiler_params=pltpu.CompilerParams(
            dimension_semantics=("arbitrary",)),
    )(x, Wg, bg.reshape(1, _E))


def _dispatch_body(xw, e1, e2, r1, r2, cnts, xg, pos, bexp, nact,
                   cnts_v, e_v, r_v, idx_v, chunk_v,
                   work_v, nact_v, sem):
    wid = lax.axis_index("s") * 2 + lax.axis_index("c")
    tok0 = wid * _TPW

    pltpu.sync_copy(cnts, cnts_v)
    c = cnts_v[...]
    lanes = lax.iota(jnp.int32, 16)
    zeros = jnp.full((16,), 0, jnp.int32)
    ones = jnp.full((16,), 1, jnp.int32)
    nb = lax.div(c + jnp.full((16,), _BLK - 1, jnp.int32),
                 jnp.full((16,), _BLK, jnp.int32))
    incl = zeros
    for e in range(_E):
        nbv = _vgather(nb, jnp.full((16,), e, jnp.int32))
        incl = incl + jnp.where(lanes >= e, nbv, zeros)
    base = (incl - nb) * jnp.full((16,), _BLK, jnp.int32)

    @pl.when(wid == 0)
    def _bexp():
        for ci in range(4):
            ivec = jnp.full((16,), ci * 16, jnp.int32) + lanes
            acc = zeros
            for e in range(_E):
                thr = _vgather(incl, jnp.full((16,), e, jnp.int32))
                acc = acc + jnp.where(ivec >= thr, ones, zeros)
            work_v[pl.ds(ci * 16, 16)] = acc
        pltpu.sync_copy(work_v, bexp)
        nact_v[...] = _vgather(incl, jnp.full((16,), 7, jnp.int32))
        pltpu.sync_copy(nact_v.at[pl.ds(0, 8)], nact)

    for k, (e_hbm, r_hbm) in enumerate(((e1, r1), (e2, r2))):
        pltpu.sync_copy(e_hbm.at[pl.ds(tok0, _TPW)], e_v)
        pltpu.sync_copy(r_hbm.at[pl.ds(tok0, _TPW)], r_v)

        for g in range(_TPW // 16):
            ev = e_v[pl.ds(g * 16, 16)]
            rv = r_v[pl.ds(g * 16, 16)]
            bv = _vgather(base, ev)
            idx_v[g // 4, pl.ds((g % 4) * 16, 16)] = bv + rv

        for h in range(_TPW // _CH):
            pltpu.sync_copy(idx_v.at[h],
                            pos.at[k, pl.ds(tok0 + h * _CH, _CH)])
            pltpu.sync_copy(xw.at[k, pl.ds(tok0 + h * _CH, _CH)], chunk_v)
            pltpu.async_copy(chunk_v, xg.at[idx_v.at[h]], sem).wait()


def _dispatch(xw, e1, e2, r1, r2, cnts):
    mesh = plsc.VectorSubcoreMesh(core_axis_name="c", subcore_axis_name="s")
    f = pl.kernel(
        _dispatch_body,
        mesh=mesh,
        out_type=[
            jax.ShapeDtypeStruct((_R_MAX, _DAUG), jnp.float32),
            jax.ShapeDtypeStruct((2, _N), jnp.int32),
            jax.ShapeDtypeStruct((64,), jnp.int32),
            jax.ShapeDtypeStruct((8,), jnp.int32),
        ],
        scratch_types=[
            pltpu.VMEM((16,), jnp.int32),
            pltpu.VMEM((_TPW,), jnp.int32),
            pltpu.VMEM((_TPW,), jnp.int32),
            pltpu.VMEM((_TPW // _CH, _CH), jnp.int32),
            pltpu.VMEM((_CH, _DAUG), jnp.float32),
            pltpu.VMEM((64,), jnp.int32),
            pltpu.VMEM((16,), jnp.int32),
            pltpu.SemaphoreType.DMA,
        ],
    )
    return f(xw, e1, e2, r1, r2, cnts)


def _gmm_body(bexp_ref, nact_ref, xg_ref, We_ref, y_ref):
    i = pl.program_id(0)

    @pl.when(i < nact_ref[0])
    def _go():
        y_ref[...] = lax.dot_general(
            xg_ref[...].astype(jnp.bfloat16), We_ref[0],
            (((1,), (0,)), ((), ())),
            preferred_element_type=jnp.float32)


def _gmm(bexp, nact, xg, We_aug):
    grid_spec = pltpu.PrefetchScalarGridSpec(
        num_scalar_prefetch=2,
        grid=(_NB_MAX,),
        in_specs=[
            pl.BlockSpec((_BLK, _DAUG), lambda i, bexp, nact: (i, 0)),
            pl.BlockSpec((1, _DAUG, _D),
                         lambda i, bexp, nact: (bexp[i], 0, 0)),
        ],
        out_specs=pl.BlockSpec((_BLK, _D), lambda i, bexp, nact: (i, 0)),
    )
    return pl.pallas_call(
        _gmm_body,
        grid_spec=grid_spec,
        out_shape=jax.ShapeDtypeStruct((_R_MAX, _D), jnp.float32),
        compiler_params=pltpu.CompilerParams(
            dimension_semantics=("arbitrary",)),
    )(bexp, nact, xg, We_aug)


def _combine_body(y, pos, out, p0_v, p1_v, y0_v, y1_v, sem):
    wid = lax.axis_index("s") * 2 + lax.axis_index("c")
    tok0 = wid * _TPW
    nch = _TPW // _CH

    pltpu.sync_copy(pos.at[0, pl.ds(tok0, _TPW)], p0_v)
    pltpu.sync_copy(pos.at[1, pl.ds(tok0, _TPW)], p1_v)

    for h in range(nch):
        pltpu.async_copy(y.at[p0_v.at[pl.ds(h * _CH, _CH)]], y0_v,
                         sem).wait()
        pltpu.async_copy(y.at[p1_v.at[pl.ds(h * _CH, _CH)]], y1_v,
                         sem).wait()

        def add_row(j, _):
            for cvec in range(_D // 16):
                o = cvec * 16
                y0_v[j, pl.ds(o, 16)] = (y0_v[j, pl.ds(o, 16)] +
                                         y1_v[j, pl.ds(o, 16)])
            return _
        lax.fori_loop(0, _CH, add_row, None)
        pltpu.sync_copy(y0_v, out.at[pl.ds(tok0 + h * _CH, _CH)])


def _combine(y, pos):
    mesh = plsc.VectorSubcoreMesh(core_axis_name="c", subcore_axis_name="s")
    f = pl.kernel(
        _combine_body,
        mesh=mesh,
        out_type=jax.ShapeDtypeStruct((_N, _D), jnp.float32),
        scratch_types=[
            pltpu.VMEM((_TPW,), jnp.int32),
            pltpu.VMEM((_TPW,), jnp.int32),
            pltpu.VMEM((_CH, _D), jnp.float32),
            pltpu.VMEM((_CH, _D), jnp.float32),
            pltpu.SemaphoreType.DMA,
        ],
    )
    return f(y, pos)


def kernel(x, Wg, bg, We, be):
    n, d = x.shape
    xw, e1, e2, r1, r2, cnts, aux = _gate(x, Wg, bg)
    xg, pos, bexp, nact = _dispatch(
        xw, e1.reshape(n), e2.reshape(n), r1.reshape(n), r2.reshape(n),
        cnts.reshape(16))
    We_aug = jnp.concatenate(
        [We, be[:, None, :],
         jnp.zeros((_E, _DAUG - _D - 1, d), We.dtype)],
        axis=1).astype(jnp.bfloat16)
    y = _gmm(bexp, nact, xg, We_aug)
    out = _combine(y, pos)
    return out, aux[0, 0]


# speedup vs baseline: 2.0696x; 2.0696x over previous
import functools

import jax
import jax.numpy as jnp
from jax.experimental import pallas as pl
from jax.experimental.pallas import tpu as pltpu

_E = 8
_NEG_INF = -1e30


def _gate_body(x_ref, Wg_ref, bg_ref, w_ref, aux_ref, probs_acc, cnt_acc,
               *, blk, n_tokens):
    t = pl.program_id(0)
    nt = pl.num_programs(0)

    logits = jax.lax.dot_general(
        x_ref[...], Wg_ref[...], (((1,), (0,)), ((), ())),
        preferred_element_type=jnp.float32) + bg_ref[...]
    iota_e = jax.lax.broadcasted_iota(jnp.int32, (blk, _E), 1)
    max1 = jnp.max(logits, axis=1, keepdims=True)
    idx1 = jnp.min(jnp.where(logits == max1, iota_e, _E), axis=1,
                   keepdims=True)
    masked = jnp.where(iota_e == idx1, _NEG_INF, logits)
    max2 = jnp.max(masked, axis=1, keepdims=True)
    idx2 = jnp.min(jnp.where(masked == max2, iota_e, _E), axis=1,
                   keepdims=True)
    e2 = jnp.exp(max2 - max1)
    w1 = 1.0 / (1.0 + e2)
    w2 = 1.0 - w1
    w_ref[...] = (jnp.where(iota_e == idx1, w1, 0.0) +
                  jnp.where(iota_e == idx2, w2, 0.0))

    probs = jnp.exp(logits - max1)
    probs = probs / jnp.sum(probs, axis=1, keepdims=True)
    block_probs = jnp.sum(probs, axis=0, keepdims=True)
    block_cnt = jnp.sum((iota_e == idx1).astype(jnp.float32), axis=0,
                        keepdims=True)

    @pl.when(t == 0)
    def _init():
        probs_acc[...] = block_probs
        cnt_acc[...] = block_cnt

    @pl.when(t > 0)
    def _accum():
        probs_acc[...] += block_probs
        cnt_acc[...] += block_cnt

    @pl.when(t == nt - 1)
    def _aux():
        aux_ref[...] = jnp.sum(
            cnt_acc[...] / (n_tokens + 1e-8) * (probs_acc[...] / n_tokens),
            axis=1, keepdims=True) * _E


def _expert_body(x_ref, w_ref, WeK_ref, be_ref, out_ref):
    xb = x_ref[...]
    w_dense = w_ref[...]
    xw = jnp.concatenate(
        [(xb * w_dense[:, e:e + 1]).astype(jnp.bfloat16) for e in range(_E)],
        axis=1)
    acc = jax.lax.dot_general(
        xw, WeK_ref[...], (((1,), (0,)), ((), ())),
        preferred_element_type=jnp.float32)
    bias = jax.lax.dot_general(
        w_dense, be_ref[...], (((1,), (0,)), ((), ())),
        preferred_element_type=jnp.float32)
    out_ref[...] = acc + bias


def kernel(x, Wg, bg, We, be):
    n, d = x.shape
    gblk = 4096
    gate_body = functools.partial(_gate_body, blk=gblk, n_tokens=n)
    w_dense, aux = pl.pallas_call(
        gate_body,
        grid=(n // gblk,),
        in_specs=[
            pl.BlockSpec((gblk, d), lambda t: (t, 0)),
            pl.BlockSpec((d, _E), lambda t: (0, 0)),
            pl.BlockSpec((1, _E), lambda t: (0, 0)),
        ],
        out_specs=[
            pl.BlockSpec((gblk, _E), lambda t: (t, 0)),
            pl.BlockSpec((1, 1), lambda t: (0, 0)),
        ],
        out_shape=[
            jax.ShapeDtypeStruct((n, _E), jnp.float32),
            jax.ShapeDtypeStruct((1, 1), jnp.float32),
        ],
        scratch_shapes=[
            pltpu.VMEM((1, _E), jnp.float32),
            pltpu.VMEM((1, _E), jnp.float32),
        ],
        compiler_params=pltpu.CompilerParams(
            dimension_semantics=("arbitrary",)),
    )(x, Wg, bg.reshape(1, _E))

    blk = 1024
    out = pl.pallas_call(
        _expert_body,
        grid=(n // blk,),
        in_specs=[
            pl.BlockSpec((blk, d), lambda t: (t, 0)),
            pl.BlockSpec((blk, _E), lambda t: (t, 0)),
            pl.BlockSpec((_E * d, d), lambda t: (0, 0)),
            pl.BlockSpec((_E, d), lambda t: (0, 0)),
        ],
        out_specs=pl.BlockSpec((blk, d), lambda t: (t, 0)),
        out_shape=jax.ShapeDtypeStruct((n, d), jnp.float32),
        compiler_params=pltpu.CompilerParams(
            dimension_semantics=("arbitrary",)),
    )(x, w_dense, We.reshape(_E * d, d).astype(jnp.bfloat16), be)
    return out, aux[0, 0]


# speedup vs baseline: 2.2002x; 1.0631x over previous
import functools

import jax
import jax.numpy as jnp
from jax.experimental import pallas as pl
from jax.experimental.pallas import tpu as pltpu

_E = 8
_NEG_INF = -1e30


def _gate_body(x_ref, Wg_ref, bg_ref, w_ref, aux_ref, probs_acc, cnt_acc,
               *, blk, n_tokens):
    t = pl.program_id(0)
    nt = pl.num_programs(0)

    logits = jax.lax.dot_general(
        x_ref[...], Wg_ref[...], (((1,), (0,)), ((), ())),
        preferred_element_type=jnp.float32) + bg_ref[...]
    iota_e = jax.lax.broadcasted_iota(jnp.int32, (blk, _E), 1)
    max1 = jnp.max(logits, axis=1, keepdims=True)
    idx1 = jnp.min(jnp.where(logits == max1, iota_e, _E), axis=1,
                   keepdims=True)
    masked = jnp.where(iota_e == idx1, _NEG_INF, logits)
    max2 = jnp.max(masked, axis=1, keepdims=True)
    idx2 = jnp.min(jnp.where(masked == max2, iota_e, _E), axis=1,
                   keepdims=True)
    e2 = jnp.exp(max2 - max1)
    w1 = 1.0 / (1.0 + e2)
    w2 = 1.0 - w1
    w_ref[...] = (jnp.where(iota_e == idx1, w1, 0.0) +
                  jnp.where(iota_e == idx2, w2, 0.0))

    probs = jnp.exp(logits - max1)
    probs = probs / jnp.sum(probs, axis=1, keepdims=True)
    block_probs = jnp.sum(probs, axis=0, keepdims=True)
    block_cnt = jnp.sum((iota_e == idx1).astype(jnp.float32), axis=0,
                        keepdims=True)

    @pl.when(t == 0)
    def _init():
        probs_acc[...] = block_probs
        cnt_acc[...] = block_cnt

    @pl.when(t > 0)
    def _accum():
        probs_acc[...] += block_probs
        cnt_acc[...] += block_cnt

    @pl.when(t == nt - 1)
    def _aux():
        aux_ref[...] = jnp.sum(
            cnt_acc[...] / (n_tokens + 1e-8) * (probs_acc[...] / n_tokens),
            axis=1, keepdims=True) * _E


def _expert_body(x_ref, w_ref, WeK_ref, be_ref, out_ref, WeK_bf):
    @pl.when(pl.program_id(0) == 0)
    def _cast():
        WeK_bf[...] = WeK_ref[...].astype(jnp.bfloat16)

    xb = x_ref[...]
    w_dense = w_ref[...]
    xw = jnp.concatenate(
        [(xb * w_dense[:, e:e + 1]).astype(jnp.bfloat16) for e in range(_E)],
        axis=1)
    acc = jax.lax.dot_general(
        xw, WeK_bf[...], (((1,), (0,)), ((), ())),
        preferred_element_type=jnp.float32)
    bias = jax.lax.dot_general(
        w_dense, be_ref[...], (((1,), (0,)), ((), ())),
        preferred_element_type=jnp.float32)
    out_ref[...] = acc + bias


def kernel(x, Wg, bg, We, be):
    n, d = x.shape
    gblk = 4096
    gate_body = functools.partial(_gate_body, blk=gblk, n_tokens=n)
    w_dense, aux = pl.pallas_call(
        gate_body,
        grid=(n // gblk,),
        in_specs=[
            pl.BlockSpec((gblk, d), lambda t: (t, 0)),
            pl.BlockSpec((d, _E), lambda t: (0, 0)),
            pl.BlockSpec((1, _E), lambda t: (0, 0)),
        ],
        out_specs=[
            pl.BlockSpec((gblk, _E), lambda t: (t, 0)),
            pl.BlockSpec((1, 1), lambda t: (0, 0)),
        ],
        out_shape=[
            jax.ShapeDtypeStruct((n, _E), jnp.float32),
            jax.ShapeDtypeStruct((1, 1), jnp.float32),
        ],
        scratch_shapes=[
            pltpu.VMEM((1, _E), jnp.float32),
            pltpu.VMEM((1, _E), jnp.float32),
        ],
        compiler_params=pltpu.CompilerParams(
            dimension_semantics=("arbitrary",)),
    )(x, Wg, bg.reshape(1, _E))

    blk = 1024
    out = pl.pallas_call(
        _expert_body,
        grid=(n // blk,),
        in_specs=[
            pl.BlockSpec((blk, d), lambda t: (t, 0)),
            pl.BlockSpec((blk, _E), lambda t: (t, 0)),
            pl.BlockSpec((_E * d, d), lambda t: (0, 0)),
            pl.BlockSpec((_E, d), lambda t: (0, 0)),
        ],
        out_specs=pl.BlockSpec((blk, d), lambda t: (t, 0)),
        out_shape=jax.ShapeDtypeStruct((n, d), jnp.float32),
        scratch_shapes=[
            pltpu.VMEM((_E * d, d), jnp.bfloat16),
        ],
        compiler_params=pltpu.CompilerParams(
            dimension_semantics=("arbitrary",)),
    )(x, w_dense, We.reshape(_E * d, d), be)
    return out, aux[0, 0]
